# Initial kernel scaffold; baseline (speedup 1.0000x reference)
import jax
import jax.numpy as jnp
from jax.experimental import pallas as pl


def kernel(x, edge_index, edge_weight, W1, b1, W2, b2):
    raise NotImplementedError("write your pallas kernel here")


# speedup vs baseline: 29.0646x; 29.0646x over previous
import functools

import jax
import jax.numpy as jnp
from jax import lax
from jax.experimental import pallas as pl
from jax.experimental.pallas import tpu as pltpu
from jax.experimental.pallas import tpu_sc as plsc

N = 10000
E = 320000
IN_CH = 128
HID = 16
OUT_CH = 64

NC = 2
NS = 16
NW = NC * NS
NPAD = 10240
NP = NPAD // NS

WE = E // NW
DC = 100
DN = WE // DC
PC = 80
PN = WE // PC

_F32 = jnp.float32
_I32 = jnp.int32


def _zero_rows(ref, n):
    z = jnp.zeros((HID,), _F32)

    @pl.loop(0, n)
    def _(i):
        ref[i] = z


def _sc_deg(dst_c, ew_c):
    mesh = plsc.VectorSubcoreMesh(core_axis_name="c", subcore_axis_name="s")

    @functools.partial(
        pl.kernel,
        out_type=jax.ShapeDtypeStruct((NC, NPAD), _F32),
        mesh=mesh,
        compiler_params=pltpu.CompilerParams(needs_layout_passes=False, use_tc_tiling_on_sc=False),
        scratch_types=[
            pltpu.VMEM_SHARED((NPAD,), _F32),
            pltpu.VMEM((DN, DC), _I32),
            pltpu.VMEM((DN, DC), _F32),
            pltpu.VMEM((NP,), _F32),
        ],
    )
    def k(dst_h, ew_h, deg_out, spmem_deg, vdst, vew, vslice):
        c = lax.axis_index("c")
        s = lax.axis_index("s")
        row0 = s * NP

        z16 = jnp.zeros((16,), _F32)
        for kk in range(NP // 16):
            vslice[pl.ds(16 * kk, 16)] = z16
        pltpu.sync_copy(vslice, spmem_deg.at[pl.ds(row0, NP)])
        plsc.subcore_barrier()

        w = c * NS + s
        pltpu.sync_copy(dst_h.at[w], vdst)
        pltpu.sync_copy(ew_h.at[w], vew)

        @pl.loop(0, DN)
        def _(j):
            pltpu.sync_copy(vew.at[j], spmem_deg.at[vdst.at[j]], add=True)

        plsc.subcore_barrier()

        pltpu.sync_copy(spmem_deg.at[pl.ds(row0, NP)], vslice)
        pltpu.sync_copy(vslice, deg_out.at[c, pl.ds(row0, NP)])

    return k(dst_c, ew_c)


def _sc_prop(hp, src_p, dst_p, ew_p):
    mesh = plsc.VectorSubcoreMesh(core_axis_name="c", subcore_axis_name="s")

    @functools.partial(
        pl.kernel,
        out_type=jax.ShapeDtypeStruct((NC, NPAD, HID), _F32),
        mesh=mesh,
        compiler_params=pltpu.CompilerParams(needs_layout_passes=False, use_tc_tiling_on_sc=False),
        scratch_types=[
            pltpu.VMEM_SHARED((NPAD, HID), _F32),
            pltpu.VMEM_SHARED((NPAD, HID), _F32),
            pltpu.VMEM((PN, PC), _I32),
            pltpu.VMEM((PN, PC), _I32),
            pltpu.VMEM((WE,), _F32),
            pltpu.VMEM((PC, HID), _F32),
            pltpu.VMEM((NP, HID), _F32),
        ],
    )
    def k(hp_h, src_h, dst_h, ew_h, agg_out,
          spmem_hp, spmem_agg, vsrc, vdst, vew, vrows, vnode):
        c = lax.axis_index("c")
        s = lax.axis_index("s")
        row0 = s * NP

        pltpu.sync_copy(hp_h.at[pl.ds(row0, NP)], vnode)
        pltpu.sync_copy(vnode, spmem_hp.at[pl.ds(row0, NP)])

        @pl.when(c == 0)
        def _():
            pltpu.sync_copy(vnode, spmem_agg.at[pl.ds(row0, NP)])

        @pl.when(c == 1)
        def _():
            _zero_rows(vnode, NP)
            pltpu.sync_copy(vnode, spmem_agg.at[pl.ds(row0, NP)])

        plsc.subcore_barrier()

        w = c * NS + s
        pltpu.sync_copy(src_h.at[w], vsrc)
        pltpu.sync_copy(dst_h.at[w], vdst)
        pltpu.sync_copy(ew_h.at[w], vew)

        @pl.loop(0, PN)
        def _(j):
            pltpu.sync_copy(spmem_hp.at[vsrc.at[j]], vrows)
            base = j * PC

            @pl.loop(0, PC)
            def _(i):
                ews = plsc.load_gather(
                    vew, [jnp.full((16,), base + i, _I32)])
                vrows[i] = vrows[i] * ews

            pltpu.sync_copy(vrows, spmem_agg.at[vdst.at[j]], add=True)

        plsc.subcore_barrier()

        pltpu.sync_copy(spmem_agg.at[pl.ds(row0, NP)], vnode)
        pltpu.sync_copy(vnode, agg_out.at[c, pl.ds(row0, NP)])

    return k(hp, src_p, dst_p, ew_p)


def _tc_pre(xp, W1, degp):
    def body(x_ref, w_ref, d0_ref, d1_ref, dis_ref, hp_ref):
        dis = lax.rsqrt(d0_ref[...] + d1_ref[...] + 1.0)
        dis_ref[...] = dis
        hp_ref[...] = dis * jnp.dot(x_ref[...], w_ref[...],
                                    preferred_element_type=_F32)

    return pl.pallas_call(
        body,
        out_shape=(
            jax.ShapeDtypeStruct((NPAD, 1), _F32),
            jax.ShapeDtypeStruct((NPAD, HID), _F32),
        ),
    )(xp, W1, degp[0].reshape(NPAD, 1), degp[1].reshape(NPAD, 1))


def _tc_mid(p, dis_col, b1_row):
    def body(p0_ref, p1_ref, dis_ref, b_ref, o_ref):
        d = dis_ref[...]
        o_ref[...] = d * jnp.maximum(
            d * (p0_ref[...] + p1_ref[...]) + b_ref[...], 0.0)

    return pl.pallas_call(
        body,
        out_shape=jax.ShapeDtypeStruct((NPAD, HID), _F32),
    )(p[0], p[1], dis_col, b1_row)


def _tc_fin(q, dis_col, W2, b2_row):
    def body(q0_ref, q1_ref, dis_ref, w_ref, b_ref, o_ref):
        t = (q0_ref[...] + q1_ref[...]) * dis_ref[...]
        sv = jnp.dot(t, w_ref[...], preferred_element_type=_F32) + b_ref[...]
        m = jnp.max(sv, axis=1, keepdims=True)
        lse = jnp.log(jnp.sum(jnp.exp(sv - m), axis=1, keepdims=True)) + m
        o_ref[...] = sv - lse

    return pl.pallas_call(
        body,
        out_shape=jax.ShapeDtypeStruct((NPAD, OUT_CH), _F32),
    )(q[0], q[1], dis_col, W2, b2_row)


def kernel(x, edge_index, edge_weight, W1, b1, W2, b2):
    src = edge_index[0].astype(_I32)
    dst = edge_index[1].astype(_I32)
    ew = edge_weight.astype(_F32)

    dst_c = dst.reshape(NW, DN, DC)
    ew_c = ew.reshape(NW, DN, DC)
    src_p = src.reshape(NW, PN, PC)
    dst_p = dst.reshape(NW, PN, PC)
    ew_p = ew.reshape(NW, WE)

    xp = jnp.pad(x, ((0, NPAD - N), (0, 0)))

    degp = _sc_deg(dst_c, ew_c)
    dis_col, h1p = _tc_pre(xp, W1, degp)
    p = _sc_prop(h1p, src_p, dst_p, ew_p)
    r1p = _tc_mid(p, dis_col, b1.reshape(1, HID))
    q = _sc_prop(r1p, src_p, dst_p, ew_p)
    out = _tc_fin(q, dis_col, W2, b2.reshape(1, OUT_CH))
    return out[:N]


# speedup vs baseline: 38.8004x; 1.3350x over previous
import functools

import jax
import jax.numpy as jnp
from jax import lax
from jax.experimental import pallas as pl
from jax.experimental.pallas import tpu as pltpu
from jax.experimental.pallas import tpu_sc as plsc

N = 10000
E = 320000
IN_CH = 128
HID = 16
OUT_CH = 64

NC = 2
NS = 16
NW = NC * NS
NPAD = 10240
NP = NPAD // NS

WE = E // NW
DC = 100
DN = WE // DC
PC = 100
PN = WE // PC

_F32 = jnp.float32
_I32 = jnp.int32


def _zero_rows(ref, n):
    z = jnp.zeros((HID,), _F32)

    @pl.loop(0, n)
    def _(i):
        ref[i] = z


def _sc_deg(dst_c, ew_c):
    mesh = plsc.VectorSubcoreMesh(core_axis_name="c", subcore_axis_name="s")

    @functools.partial(
        pl.kernel,
        out_type=jax.ShapeDtypeStruct((NC, NPAD), _F32),
        mesh=mesh,
        compiler_params=pltpu.CompilerParams(needs_layout_passes=False, use_tc_tiling_on_sc=False),
        scratch_types=[
            pltpu.VMEM_SHARED((NPAD,), _F32),
            pltpu.VMEM((DN, DC), _I32),
            pltpu.VMEM((DN, DC), _F32),
            pltpu.VMEM((NP,), _F32),
        ],
    )
    def k(dst_h, ew_h, deg_out, spmem_deg, vdst, vew, vslice):
        c = lax.axis_index("c")
        s = lax.axis_index("s")
        row0 = s * NP

        z16 = jnp.zeros((16,), _F32)
        for kk in range(NP // 16):
            vslice[pl.ds(16 * kk, 16)] = z16
        pltpu.sync_copy(vslice, spmem_deg.at[pl.ds(row0, NP)])
        plsc.subcore_barrier()

        w = c * NS + s
        pltpu.sync_copy(dst_h.at[w], vdst)
        pltpu.sync_copy(ew_h.at[w], vew)

        @pl.loop(0, DN)
        def _(j):
            pltpu.sync_copy(vew.at[j], spmem_deg.at[vdst.at[j]], add=True)

        plsc.subcore_barrier()

        pltpu.sync_copy(spmem_deg.at[pl.ds(row0, NP)], vslice)
        pltpu.sync_copy(vslice, deg_out.at[c, pl.ds(row0, NP)])

    return k(dst_c, ew_c)


def _make_sc_prop(fuse_mid):
    mesh = plsc.VectorSubcoreMesh(core_axis_name="c", subcore_axis_name="s")

    scratch = [
        pltpu.VMEM_SHARED((NPAD, HID), _F32),
        pltpu.VMEM_SHARED((NPAD, HID), _F32),
        pltpu.VMEM((PN + 2, PC), _I32),
        pltpu.VMEM((PN, PC), _I32),
        pltpu.VMEM((WE,), _F32),
        pltpu.VMEM((PC, HID), _F32),
        pltpu.VMEM((PC, HID), _F32),
        pltpu.VMEM((PC, HID), _F32),
        pltpu.VMEM((PC, HID), _F32),
        pltpu.VMEM((NP, HID), _F32),
        pltpu.SemaphoreType.DMA,
        pltpu.SemaphoreType.DMA,
        pltpu.SemaphoreType.DMA,
        pltpu.SemaphoreType.DMA,
    ]
    if fuse_mid:
        scratch += [
            pltpu.VMEM((NP, HID), _F32),
            pltpu.VMEM((NP, HID), _F32),
            pltpu.VMEM((NP, HID), _F32),
            pltpu.VMEM((1, HID), _F32),
        ]

    def body(refs):
        if fuse_mid:
            (p_h, disb_h, b1_h, src_h, dst_h, ew_h, agg_out,
             spmem_hp, spmem_agg, vsrc, vdst, vew,
             g0, g1, s0, s1, vnode, sg0, sg1, ss0, ss1,
             vp0, vp1, vdis, vb1) = refs
        else:
            (hp_h, src_h, dst_h, ew_h, agg_out,
             spmem_hp, spmem_agg, vsrc, vdst, vew,
             g0, g1, s0, s1, vnode, sg0, sg1, ss0, ss1) = refs

        c = lax.axis_index("c")
        s = lax.axis_index("s")
        row0 = s * NP

        if fuse_mid:
            pltpu.sync_copy(p_h.at[0, pl.ds(row0, NP)], vp0)
            pltpu.sync_copy(p_h.at[1, pl.ds(row0, NP)], vp1)
            pltpu.sync_copy(disb_h.at[pl.ds(row0, NP)], vdis)
            pltpu.sync_copy(b1_h, vb1)
            bv = vb1[0]

            @pl.loop(0, NP)
            def _(i):
                d = vdis[i]
                vnode[i] = d * jnp.maximum(
                    d * (vp0[i] + vp1[i]) + bv, 0.0)
        else:
            pltpu.sync_copy(hp_h.at[pl.ds(row0, NP)], vnode)

        pltpu.sync_copy(vnode, spmem_hp.at[pl.ds(row0, NP)])

        @pl.when(c == 0)
        def _():
            pltpu.sync_copy(vnode, spmem_agg.at[pl.ds(row0, NP)])

        @pl.when(c == 1)
        def _():
            _zero_rows(vnode, NP)
            pltpu.sync_copy(vnode, spmem_agg.at[pl.ds(row0, NP)])

        plsc.subcore_barrier()

        w = c * NS + s
        pltpu.sync_copy(src_h.at[w], vsrc)
        pltpu.sync_copy(dst_h.at[w], vdst)
        pltpu.sync_copy(ew_h.at[w], vew)

        bufs = ((g0, s0, sg0, ss0), (g1, s1, sg1, ss1))

        def scale(gb, sb, j):
            base = j * PC

            @pl.loop(0, PC)
            def _(i):
                ews = plsc.load_gather(
                    vew, [jnp.full((16,), base + i, _I32)])
                sb[i] = gb[i] * ews

        pltpu.async_copy(spmem_hp.at[vsrc.at[0]], g0, sg0)
        pltpu.async_copy(spmem_hp.at[vsrc.at[1]], g1, sg1)

        for b in range(2):
            gb, sb, sg, ss = bufs[b]
            pltpu.make_async_copy(spmem_hp.at[vsrc.at[b]], gb, sg).wait()
            scale(gb, sb, b)
            pltpu.async_copy(sb, spmem_agg.at[vdst.at[b]], ss, add=True)
            pltpu.async_copy(spmem_hp.at[vsrc.at[b + 2]], gb, sg)

        @pl.loop(1, PN // 2)
        def _(p):
            for b in range(2):
                gb, sb, sg, ss = bufs[b]
                j = 2 * p + b
                pltpu.make_async_copy(
                    spmem_hp.at[vsrc.at[j]], gb, sg).wait()
                pltpu.make_async_copy(
                    sb, spmem_agg.at[vdst.at[j]], ss).wait()
                scale(gb, sb, j)
                pltpu.async_copy(sb, spmem_agg.at[vdst.at[j]], ss, add=True)
                pltpu.async_copy(spmem_hp.at[vsrc.at[j + 2]], gb, sg)

        for b in range(2):
            gb, sb, sg, ss = bufs[b]
            pltpu.make_async_copy(
                sb, spmem_agg.at[vdst.at[PN - 2 + b]], ss).wait()
            pltpu.make_async_copy(
                spmem_hp.at[vsrc.at[PN + b]], gb, sg).wait()

        plsc.subcore_barrier()

        pltpu.sync_copy(spmem_agg.at[pl.ds(row0, NP)], vnode)
        pltpu.sync_copy(vnode, agg_out.at[c, pl.ds(row0, NP)])

    if fuse_mid:
        def k(p_h, disb_h, b1_h, src_h, dst_h, ew_h, agg_out, *scr):
            body((p_h, disb_h, b1_h, src_h, dst_h, ew_h, agg_out) + scr)
    else:
        def k(hp_h, src_h, dst_h, ew_h, agg_out, *scr):
            body((hp_h, src_h, dst_h, ew_h, agg_out) + scr)

    return functools.partial(
        pl.kernel,
        out_type=jax.ShapeDtypeStruct((NC, NPAD, HID), _F32),
        mesh=mesh,
        compiler_params=pltpu.CompilerParams(needs_layout_passes=False, use_tc_tiling_on_sc=False),
        scratch_types=scratch,
    )(k)


def _tc_pre(xp, W1, degp):
    def body(x_ref, w_ref, d0_ref, d1_ref, disb_ref, hp_ref):
        dis = lax.rsqrt(d0_ref[...] + d1_ref[...] + 1.0)
        disb = jnp.broadcast_to(dis, (NPAD, HID))
        disb_ref[...] = disb
        hp_ref[...] = disb * jnp.dot(x_ref[...], w_ref[...],
                                     preferred_element_type=_F32)

    return pl.pallas_call(
        body,
        out_shape=(
            jax.ShapeDtypeStruct((NPAD, HID), _F32),
            jax.ShapeDtypeStruct((NPAD, HID), _F32),
        ),
    )(xp, W1, degp[0].reshape(NPAD, 1), degp[1].reshape(NPAD, 1))


def _tc_fin(q, disb, W2, b2_row):
    def body(q0_ref, q1_ref, disb_ref, w_ref, b_ref, o_ref):
        t = (q0_ref[...] + q1_ref[...]) * disb_ref[...]
        sv = jnp.dot(t, w_ref[...], preferred_element_type=_F32) + b_ref[...]
        m = jnp.max(sv, axis=1, keepdims=True)
        lse = jnp.log(jnp.sum(jnp.exp(sv - m), axis=1, keepdims=True)) + m
        o_ref[...] = sv - lse

    return pl.pallas_call(
        body,
        out_shape=jax.ShapeDtypeStruct((NPAD, OUT_CH), _F32),
    )(q[0], q[1], disb, W2, b2_row)


def kernel(x, edge_index, edge_weight, W1, b1, W2, b2):
    src = edge_index[0].astype(_I32)
    dst = edge_index[1].astype(_I32)
    ew = edge_weight.astype(_F32)

    dst_c = dst.reshape(NW, DN, DC)
    ew_c = ew.reshape(NW, DN, DC)
    src_p = jnp.pad(src.reshape(NW, PN, PC), ((0, 0), (0, 2), (0, 0)))
    dst_p = dst.reshape(NW, PN, PC)
    ew_p = ew.reshape(NW, WE)

    xp = jnp.pad(x, ((0, NPAD - N), (0, 0)))

    degp = _sc_deg(dst_c, ew_c)
    disb, h1p = _tc_pre(xp, W1, degp)
    p = _make_sc_prop(False)(h1p, src_p, dst_p, ew_p)
    q = _make_sc_prop(True)(p, disb, b1.reshape(1, HID), src_p, dst_p, ew_p)
    out = _tc_fin(q, disb, W2, b2.reshape(1, OUT_CH))
    return out[:N]
